# baseline (speedup 1.0000x reference)
import functools

import jax
import jax.numpy as jnp
from jax import lax
from jax.experimental import pallas as pl
from jax.experimental.pallas import tpu as pltpu
from jax.experimental.pallas import tpu_sc as plsc

N_NODES = 10000
N_PAD = 10112
N_EDGES = 160000
E_PAD = 160256
D_HID = 16
NC = 2
NS = 16
E_PER_TILE = E_PAD // (NC * NS)
ROWS_PER_TILE = N_PAD // NS


def _splat_lane(vec, j):
    return lax.gather(
        vec, jnp.full((D_HID, 1), j, jnp.int32),
        dimension_numbers=lax.GatherDimensionNumbers(
            offset_dims=(), collapsed_slice_dims=(0,), start_index_map=(0,)),
        slice_sizes=(1,),
        mode=lax.GatherScatterMode.PROMISE_IN_BOUNDS)


def _make_sc_spmm(fused):
    mesh = plsc.VectorSubcoreMesh(core_axis_name="c", subcore_axis_name="s")

    out_slab = jax.ShapeDtypeStruct((N_PAD, D_HID), jnp.float32)
    scratch = [
        pltpu.VMEM((E_PER_TILE,), jnp.int32),
        pltpu.VMEM((E_PER_TILE,), jnp.int32),
        pltpu.VMEM((E_PER_TILE,), jnp.float32),
        pltpu.VMEM((E_PER_TILE, D_HID), jnp.float32),
        pltpu.VMEM((ROWS_PER_TILE, D_HID), jnp.float32),
        pltpu.VMEM((D_HID,), jnp.float32),
        pltpu.VMEM_SHARED((N_PAD, D_HID), jnp.float32),
        pltpu.SemaphoreType.DMA,
    ]

    def body(refs):
        if fused:
            (rows_hbm, cols_hbm, vals_hbm, h0_hbm, h1_hbm, b_hbm,
             out0_hbm, out1_hbm,
             rows_v, cols_v, vals_v, gath_v, zbuf_v, b_v, acc_sh, sem) = refs
        else:
            (rows_hbm, cols_hbm, vals_hbm, h0_hbm,
             out0_hbm, out1_hbm,
             rows_v, cols_v, vals_v, gath_v, zbuf_v, b_v, acc_sh, sem) = refs
        c = lax.axis_index("c")
        s = lax.axis_index("s")
        base = (c * NS + s) * E_PER_TILE

        cp_rows = pltpu.async_copy(
            rows_hbm.at[pl.ds(base, E_PER_TILE)], rows_v, sem)
        cp_cols = pltpu.async_copy(
            cols_hbm.at[pl.ds(base, E_PER_TILE)], cols_v, sem)
        cp_vals = pltpu.async_copy(
            vals_hbm.at[pl.ds(base, E_PER_TILE)], vals_v, sem)

        def zero_body(j, _):
            zbuf_v[j, :] = jnp.zeros((D_HID,), jnp.float32)
            return 0
        lax.fori_loop(0, ROWS_PER_TILE, zero_body, 0)
        pltpu.sync_copy(zbuf_v,
                        acc_sh.at[pl.ds(s * ROWS_PER_TILE, ROWS_PER_TILE)])

        cp_rows.wait()
        cp_cols.wait()
        cp_vals.wait()

        pltpu.async_copy(h0_hbm.at[cols_v], gath_v, sem).wait()
        if fused:
            pltpu.async_copy(h1_hbm.at[cols_v], gath_v, sem,
                             add=True).wait()
            pltpu.sync_copy(b_hbm, b_v)
            bias = b_v[:]

        @plsc.parallel_loop(0, E_PER_TILE // 16, unroll=4)
        def scale_body(g):
            vals16 = vals_v[pl.ds(g * 16, 16)]
            for j in range(16):
                e = g * 16 + j
                row = gath_v[e, :]
                if fused:
                    row = jnp.maximum(row + bias, 0.0)
                gath_v[e, :] = row * _splat_lane(vals16, j)

        plsc.subcore_barrier()

        pltpu.sync_copy(gath_v, acc_sh.at[rows_v], add=True)

        plsc.subcore_barrier()

        acc_slice = acc_sh.at[pl.ds(s * ROWS_PER_TILE, ROWS_PER_TILE)]

        @pl.when(c == 0)
        def _():
            pltpu.sync_copy(
                acc_slice,
                out0_hbm.at[pl.ds(s * ROWS_PER_TILE, ROWS_PER_TILE)])

        @pl.when(c == 1)
        def _():
            pltpu.sync_copy(
                acc_slice,
                out1_hbm.at[pl.ds(s * ROWS_PER_TILE, ROWS_PER_TILE)])

    def entry(*refs):
        body(refs)

    return pl.kernel(
        entry,
        mesh=mesh,
        compiler_params=pltpu.CompilerParams(use_tc_tiling_on_sc=False),
        out_type=(out_slab, out_slab),
        scratch_types=scratch,
    )


_sc_spmm = _make_sc_spmm(fused=False)
_sc_spmm_fused = _make_sc_spmm(fused=True)


def _tc_matmul1(feature, w1):
    m, k = feature.shape
    n = w1.shape[1]
    bm = 632

    def body(x_ref, w_ref, o_ref):
        o_ref[:] = jnp.dot(x_ref[:], w_ref[:],
                           preferred_element_type=jnp.float32)

    return pl.pallas_call(
        body,
        grid=(m // bm,),
        in_specs=[pl.BlockSpec((bm, k), lambda i: (i, 0)),
                  pl.BlockSpec((k, n), lambda i: (0, 0))],
        out_specs=pl.BlockSpec((bm, n), lambda i: (i, 0)),
        out_shape=jax.ShapeDtypeStruct((m, n), jnp.float32),
    )(feature, w1)


def _tc_final(q0, q1, w2, b2):
    m, k = q0.shape
    n = w2.shape[1]
    bm = 632

    def body(p_ref, q_ref, w_ref, b_ref, o_ref):
        g = p_ref[:] + q_ref[:]
        o_ref[:] = jnp.dot(g, w_ref[:],
                           preferred_element_type=jnp.float32) + b_ref[:]

    return pl.pallas_call(
        body,
        grid=(m // bm,),
        in_specs=[pl.BlockSpec((bm, k), lambda i: (i, 0)),
                  pl.BlockSpec((bm, k), lambda i: (i, 0)),
                  pl.BlockSpec((k, n), lambda i: (0, 0)),
                  pl.BlockSpec((1, n), lambda i: (0, 0))],
        out_specs=pl.BlockSpec((bm, n), lambda i: (i, 0)),
        out_shape=jax.ShapeDtypeStruct((m, n), jnp.float32),
    )(q0, q1, w2, b2)


def kernel(adjacency_edge_index, adjacency_values, feature, W1, b1, W2, b2):
    epad = E_PAD - N_EDGES
    rows = jnp.pad(adjacency_edge_index[0], (0, epad))
    cols = jnp.pad(adjacency_edge_index[1], (0, epad))
    vals = jnp.pad(adjacency_values, (0, epad))
    feature_p = jnp.pad(feature, ((0, N_PAD - N_NODES), (0, 0)))

    h1 = _tc_matmul1(feature_p, W1)
    p0, p1 = _sc_spmm(rows, cols, vals, h1)
    q0, q1 = _sc_spmm_fused(rows, cols, vals, p0, p1, b1)
    return _tc_final(q0, q1, W2, b2.reshape(1, -1))[:N_NODES]


# speedup vs baseline: 1.0753x; 1.0753x over previous
import functools

import jax
import jax.numpy as jnp
from jax import lax
from jax.experimental import pallas as pl
from jax.experimental.pallas import tpu as pltpu
from jax.experimental.pallas import tpu_sc as plsc

N_NODES = 10000
N_PAD = 10112
N_EDGES = 160000
E_PAD = 160768
D_HID = 16
NC = 2
NS = 16
E_PER_TILE = E_PAD // (NC * NS)
E_HALF = E_PER_TILE // 2
ROWS_PER_TILE = N_PAD // NS


def _splat_lane(vec, j):
    return lax.gather(
        vec, jnp.full((D_HID, 1), j, jnp.int32),
        dimension_numbers=lax.GatherDimensionNumbers(
            offset_dims=(), collapsed_slice_dims=(0,), start_index_map=(0,)),
        slice_sizes=(1,),
        mode=lax.GatherScatterMode.PROMISE_IN_BOUNDS)


def _make_sc_spmm(fused):
    mesh = plsc.VectorSubcoreMesh(core_axis_name="c", subcore_axis_name="s")

    out_slab = jax.ShapeDtypeStruct((N_PAD, D_HID), jnp.float32)
    scratch = [
        pltpu.VMEM((2, E_HALF), jnp.int32),
        pltpu.VMEM((2, E_HALF), jnp.int32),
        pltpu.VMEM((2, E_HALF), jnp.float32),
        pltpu.VMEM((2, E_HALF, D_HID), jnp.float32),
        pltpu.VMEM((ROWS_PER_TILE, D_HID), jnp.float32),
        pltpu.VMEM((D_HID,), jnp.float32),
        pltpu.VMEM_SHARED((N_PAD, D_HID), jnp.float32),
        pltpu.SemaphoreType.DMA,
        pltpu.SemaphoreType.DMA,
        pltpu.SemaphoreType.DMA,
        pltpu.SemaphoreType.DMA,
    ]

    def body(refs):
        if fused:
            (rows_hbm, cols_hbm, vals_hbm, h0_hbm, h1_hbm, b_hbm,
             out0_hbm, out1_hbm,
             rows_v, cols_v, vals_v, gath_v, zbuf_v, b_v, acc_sh,
             sem_st, sem_g0, sem_g1, sem_sc) = refs
        else:
            (rows_hbm, cols_hbm, vals_hbm, h0_hbm,
             out0_hbm, out1_hbm,
             rows_v, cols_v, vals_v, gath_v, zbuf_v, b_v, acc_sh,
             sem_st, sem_g0, sem_g1, sem_sc) = refs
        c = lax.axis_index("c")
        s = lax.axis_index("s")
        base = (c * NS + s) * E_PER_TILE

        stage = []
        for h in range(2):
            hb = base + h * E_HALF
            stage.append(pltpu.async_copy(
                rows_hbm.at[pl.ds(hb, E_HALF)], rows_v.at[h], sem_st))
            stage.append(pltpu.async_copy(
                cols_hbm.at[pl.ds(hb, E_HALF)], cols_v.at[h], sem_st))
            stage.append(pltpu.async_copy(
                vals_hbm.at[pl.ds(hb, E_HALF)], vals_v.at[h], sem_st))

        def zero_body(j, _):
            zbuf_v[j, :] = jnp.zeros((D_HID,), jnp.float32)
            return 0
        lax.fori_loop(0, ROWS_PER_TILE, zero_body, 0)
        pltpu.sync_copy(zbuf_v,
                        acc_sh.at[pl.ds(s * ROWS_PER_TILE, ROWS_PER_TILE)])
        if fused:
            pltpu.sync_copy(b_hbm, b_v)
            bias = b_v[:]
        for cp in stage:
            cp.wait()

        plsc.subcore_barrier()

        def scale_half(h):
            def scale_body(g, _):
                vals16 = vals_v[h, pl.ds(g * 16, 16)]
                for j in range(16):
                    e = g * 16 + j
                    row = gath_v[h, e, :]
                    if fused:
                        row = jnp.maximum(row + bias, 0.0)
                    gath_v[h, e, :] = row * _splat_lane(vals16, j)
                return 0
            lax.fori_loop(0, E_HALF // 16, scale_body, 0)

        gA = pltpu.async_copy(h0_hbm.at[cols_v.at[0]], gath_v.at[0], sem_g0)
        gA.wait()
        gB = pltpu.async_copy(h0_hbm.at[cols_v.at[1]], gath_v.at[1], sem_g0)
        if fused:
            gA2 = pltpu.async_copy(h1_hbm.at[cols_v.at[0]], gath_v.at[0],
                                   sem_g1, add=True)
            gA2.wait()
        scale_half(0)
        scA = pltpu.async_copy(gath_v.at[0], acc_sh.at[rows_v.at[0]],
                               sem_sc, add=True)
        gB.wait()
        if fused:
            gB2 = pltpu.async_copy(h1_hbm.at[cols_v.at[1]], gath_v.at[1],
                                   sem_g1, add=True)
            gB2.wait()
        scale_half(1)
        scA.wait()
        pltpu.async_copy(gath_v.at[1], acc_sh.at[rows_v.at[1]],
                         sem_sc, add=True).wait()

        plsc.subcore_barrier()

        acc_slice = acc_sh.at[pl.ds(s * ROWS_PER_TILE, ROWS_PER_TILE)]

        @pl.when(c == 0)
        def _():
            pltpu.sync_copy(
                acc_slice,
                out0_hbm.at[pl.ds(s * ROWS_PER_TILE, ROWS_PER_TILE)])

        @pl.when(c == 1)
        def _():
            pltpu.sync_copy(
                acc_slice,
                out1_hbm.at[pl.ds(s * ROWS_PER_TILE, ROWS_PER_TILE)])

    def entry(*refs):
        body(refs)

    return pl.kernel(
        entry,
        mesh=mesh,
        compiler_params=pltpu.CompilerParams(use_tc_tiling_on_sc=False),
        out_type=(out_slab, out_slab),
        scratch_types=scratch,
    )


_sc_spmm = _make_sc_spmm(fused=False)
_sc_spmm_fused = _make_sc_spmm(fused=True)


def _tc_matmul1(feature, w1):
    m, k = feature.shape
    n = w1.shape[1]
    bm = 632

    def body(x_ref, w_ref, o_ref):
        o_ref[:] = jnp.dot(x_ref[:], w_ref[:],
                           preferred_element_type=jnp.float32)

    return pl.pallas_call(
        body,
        grid=(m // bm,),
        in_specs=[pl.BlockSpec((bm, k), lambda i: (i, 0)),
                  pl.BlockSpec((k, n), lambda i: (0, 0))],
        out_specs=pl.BlockSpec((bm, n), lambda i: (i, 0)),
        out_shape=jax.ShapeDtypeStruct((m, n), jnp.float32),
    )(feature, w1)


def _tc_final(q0, q1, w2, b2):
    m, k = q0.shape
    n = w2.shape[1]
    bm = 632

    def body(p_ref, q_ref, w_ref, b_ref, o_ref):
        g = p_ref[:] + q_ref[:]
        o_ref[:] = jnp.dot(g, w_ref[:],
                           preferred_element_type=jnp.float32) + b_ref[:]

    return pl.pallas_call(
        body,
        grid=(m // bm,),
        in_specs=[pl.BlockSpec((bm, k), lambda i: (i, 0)),
                  pl.BlockSpec((bm, k), lambda i: (i, 0)),
                  pl.BlockSpec((k, n), lambda i: (0, 0)),
                  pl.BlockSpec((1, n), lambda i: (0, 0))],
        out_specs=pl.BlockSpec((bm, n), lambda i: (i, 0)),
        out_shape=jax.ShapeDtypeStruct((m, n), jnp.float32),
    )(q0, q1, w2, b2)


def kernel(adjacency_edge_index, adjacency_values, feature, W1, b1, W2, b2):
    epad = E_PAD - N_EDGES
    rows = jnp.pad(adjacency_edge_index[0], (0, epad))
    cols = jnp.pad(adjacency_edge_index[1], (0, epad))
    vals = jnp.pad(adjacency_values, (0, epad))
    feature_p = jnp.pad(feature, ((0, N_PAD - N_NODES), (0, 0)))

    h1 = _tc_matmul1(feature_p, W1)
    p0, p1 = _sc_spmm(rows, cols, vals, h1)
    q0, q1 = _sc_spmm_fused(rows, cols, vals, p0, p1, b1)
    return _tc_final(q0, q1, W2, b2.reshape(1, -1))[:N_NODES]


# speedup vs baseline: 1.1646x; 1.0831x over previous
import functools

import jax
import jax.numpy as jnp
from jax import lax
from jax.experimental import pallas as pl
from jax.experimental.pallas import tpu as pltpu
from jax.experimental.pallas import tpu_sc as plsc

N_NODES = 10000
N_PAD = 10112
N_EDGES = 160000
E_PAD = 160256
D_HID = 16
NC = 2
NS = 16
E_PER_TILE = E_PAD // (NC * NS)
ROWS_PER_TILE = N_PAD // NS


def _splat_lane(vec, j):
    return lax.gather(
        vec, jnp.full((D_HID, 1), j, jnp.int32),
        dimension_numbers=lax.GatherDimensionNumbers(
            offset_dims=(), collapsed_slice_dims=(0,), start_index_map=(0,)),
        slice_sizes=(1,),
        mode=lax.GatherScatterMode.PROMISE_IN_BOUNDS)


def _make_sc_gcn():
    mesh = plsc.VectorSubcoreMesh(core_axis_name="c", subcore_axis_name="s")
    slab = jax.ShapeDtypeStruct((N_PAD, D_HID), jnp.float32)

    @functools.partial(
        pl.kernel,
        mesh=mesh,
        compiler_params=pltpu.CompilerParams(use_tc_tiling_on_sc=False),
        out_type=(slab, slab, slab, slab),
        scratch_types=[
            pltpu.VMEM((E_PER_TILE,), jnp.int32),
            pltpu.VMEM((E_PER_TILE,), jnp.int32),
            pltpu.VMEM((E_PER_TILE,), jnp.float32),
            pltpu.VMEM((E_PER_TILE, D_HID), jnp.float32),
            pltpu.VMEM((ROWS_PER_TILE, D_HID), jnp.float32),
            pltpu.VMEM((D_HID,), jnp.float32),
            pltpu.VMEM_SHARED((N_PAD, D_HID), jnp.float32),
            pltpu.SemaphoreType.DMA,
            pltpu.SemaphoreType.REGULAR,
        ],
    )
    def gcn(rows_hbm, cols_hbm, vals_hbm, h1_hbm, b_hbm,
            q0_hbm, q1_hbm, p0_hbm, p1_hbm,
            rows_v, cols_v, vals_v, gath_v, zbuf_v, b_v, acc_sh,
            sem, semx):
        c = lax.axis_index("c")
        s = lax.axis_index("s")
        base = (c * NS + s) * E_PER_TILE
        my_rows = pl.ds(s * ROWS_PER_TILE, ROWS_PER_TILE)

        cp_rows = pltpu.async_copy(
            rows_hbm.at[pl.ds(base, E_PER_TILE)], rows_v, sem)
        cp_cols = pltpu.async_copy(
            cols_hbm.at[pl.ds(base, E_PER_TILE)], cols_v, sem)
        cp_vals = pltpu.async_copy(
            vals_hbm.at[pl.ds(base, E_PER_TILE)], vals_v, sem)

        def zero_body(j, _):
            zbuf_v[j, :] = jnp.zeros((D_HID,), jnp.float32)
            return 0
        lax.fori_loop(0, ROWS_PER_TILE, zero_body, 0)
        pltpu.sync_copy(zbuf_v, acc_sh.at[my_rows])
        pltpu.sync_copy(b_hbm, b_v)
        bias = b_v[:]

        cp_rows.wait()
        cp_cols.wait()
        cp_vals.wait()

        def scale(fused):
            def scale_body(g, _):
                vals16 = vals_v[pl.ds(g * 16, 16)]
                for j in range(16):
                    e = g * 16 + j
                    row = gath_v[e, :]
                    if fused:
                        row = jnp.maximum(row + bias, 0.0)
                    gath_v[e, :] = row * _splat_lane(vals16, j)
                return 0
            lax.fori_loop(0, E_PER_TILE // 16, scale_body, 0)

        plsc.subcore_barrier()

        pltpu.async_copy(h1_hbm.at[cols_v], gath_v, sem).wait()
        scale(fused=False)
        pltpu.sync_copy(gath_v, acc_sh.at[rows_v], add=True)
        plsc.subcore_barrier()

        @pl.when(c == 0)
        def _():
            pltpu.sync_copy(acc_sh.at[my_rows], p0_hbm.at[my_rows])

        @pl.when(c == 1)
        def _():
            pltpu.sync_copy(acc_sh.at[my_rows], p1_hbm.at[my_rows])

        pltpu.sync_copy(zbuf_v, acc_sh.at[my_rows])
        plsc.subcore_barrier()

        pl.semaphore_signal(semx, 1, core_index=1 - c)
        pl.semaphore_wait(semx, 1)

        pltpu.async_copy(p0_hbm.at[cols_v], gath_v, sem).wait()
        pltpu.async_copy(p1_hbm.at[cols_v], gath_v, sem, add=True).wait()
        scale(fused=True)
        pltpu.sync_copy(gath_v, acc_sh.at[rows_v], add=True)
        plsc.subcore_barrier()

        @pl.when(c == 0)
        def _():
            pltpu.sync_copy(acc_sh.at[my_rows], q0_hbm.at[my_rows])

        @pl.when(c == 1)
        def _():
            pltpu.sync_copy(acc_sh.at[my_rows], q1_hbm.at[my_rows])

    return gcn


_sc_gcn = _make_sc_gcn()


def _tc_matmul1(feature, w1):
    m, k = feature.shape
    n = w1.shape[1]
    bm = 632

    def body(x_ref, w_ref, o_ref):
        o_ref[:] = jnp.dot(x_ref[:], w_ref[:],
                           preferred_element_type=jnp.float32)

    return pl.pallas_call(
        body,
        grid=(m // bm,),
        in_specs=[pl.BlockSpec((bm, k), lambda i: (i, 0)),
                  pl.BlockSpec((k, n), lambda i: (0, 0))],
        out_specs=pl.BlockSpec((bm, n), lambda i: (i, 0)),
        out_shape=jax.ShapeDtypeStruct((m, n), jnp.float32),
    )(feature, w1)


def _tc_final(q0, q1, w2, b2):
    m, k = q0.shape
    n = w2.shape[1]
    bm = 632

    def body(p_ref, q_ref, w_ref, b_ref, o_ref):
        g = p_ref[:] + q_ref[:]
        o_ref[:] = jnp.dot(g, w_ref[:],
                           preferred_element_type=jnp.float32) + b_ref[:]

    return pl.pallas_call(
        body,
        grid=(m // bm,),
        in_specs=[pl.BlockSpec((bm, k), lambda i: (i, 0)),
                  pl.BlockSpec((bm, k), lambda i: (i, 0)),
                  pl.BlockSpec((k, n), lambda i: (0, 0)),
                  pl.BlockSpec((1, n), lambda i: (0, 0))],
        out_specs=pl.BlockSpec((bm, n), lambda i: (i, 0)),
        out_shape=jax.ShapeDtypeStruct((m, n), jnp.float32),
    )(q0, q1, w2, b2)


def kernel(adjacency_edge_index, adjacency_values, feature, W1, b1, W2, b2):
    epad = E_PAD - N_EDGES
    rows = jnp.pad(adjacency_edge_index[0], (0, epad))
    cols = jnp.pad(adjacency_edge_index[1], (0, epad))
    vals = jnp.pad(adjacency_values, (0, epad))
    feature_p = jnp.pad(feature, ((0, N_PAD - N_NODES), (0, 0)))

    h1 = _tc_matmul1(feature_p, W1)
    q0, q1, _, _ = _sc_gcn(rows, cols, vals, h1, b1)
    return _tc_final(q0, q1, W2, b2.reshape(1, -1))[:N_NODES]
